# Initial kernel scaffold; baseline (speedup 1.0000x reference)
import jax
import jax.numpy as jnp
from jax.experimental import pallas as pl


def kernel(rel_pos, W_proj, qlen, klen):
    raise NotImplementedError("write your pallas kernel here")


# speedup vs baseline: 332.0052x; 332.0052x over previous
import functools

import jax
import jax.numpy as jnp
from jax import lax
from jax.experimental import pallas as pl
from jax.experimental.pallas import tpu as pltpu
from jax.experimental.pallas import tpu_sc as plsc


def _proj_body(rel_ref, w_ref, v_ref):
    s = jnp.sum(rel_ref[...] * w_ref[...], axis=1)
    v_ref[...] = jnp.concatenate([s, jnp.zeros((1,), jnp.float32)])


def _project(rel_pos, w_proj):
    n = rel_pos.shape[0]
    return pl.pallas_call(
        _proj_body,
        out_shape=jax.ShapeDtypeStruct((n + 1,), jnp.float32),
    )(rel_pos, w_proj)


def _make_expand(L, NC, NS):
    NW = NC * NS
    assert L % (8 * (NW // 8)) == 0
    rows_per_w = L // NW
    groups = NW // 8
    n_pad = 2 * L
    mesh = plsc.VectorSubcoreMesh(core_axis_name="c", subcore_axis_name="s")

    @functools.partial(
        pl.kernel,
        mesh=mesh,
        out_type=jax.ShapeDtypeStruct((L * L,), jnp.float32),
        scratch_types=[
            pltpu.VMEM((n_pad + 16,), jnp.float32),
            pltpu.VMEM((n_pad,), jnp.float32),
            pltpu.SemaphoreType.DMA,
        ],
    )
    def expand(v_hbm, out_hbm, v_raw, v_shift, sem):
        wid = lax.axis_index("s") * NC + lax.axis_index("c")
        m = wid % 8
        g = wid // 8
        pltpu.sync_copy(v_hbm, v_raw.at[pl.ds(0, n_pad)])

        r = (L - 1 - m) % 8

        def shift_body(j, _):
            v_shift[pl.ds(16 * j, 16)] = v_raw[pl.ds(r + 16 * j, 16)]
            return 0

        lax.fori_loop(0, n_pad // 16, shift_body, 0)

        copies = []
        for i in range(rows_per_w):
            k = g * rows_per_w + i
            q = m + 8 * k
            start = pl.multiple_of((L - 1 - q) - r, 8)
            dst = pl.multiple_of(q * L, 8)
            copies.append(
                pltpu.async_copy(
                    v_shift.at[pl.ds(start, L)], out_hbm.at[pl.ds(dst, L)], sem
                )
            )
        for c in copies:
            c.wait()

    return expand


def kernel(rel_pos, W_proj, qlen, klen):
    L = (rel_pos.shape[0] + 1) // 2
    v = _project(rel_pos, W_proj)
    info = plsc.get_sparse_core_info()
    expand = _make_expand(L, info.num_cores, info.num_subcores)
    return expand(v).reshape(L, L)
